# baseline (speedup 1.0000x reference)
import functools

import jax
import jax.numpy as jnp
from jax import lax
from jax.experimental import pallas as pl
from jax.experimental.pallas import tpu as pltpu
from jax.experimental.pallas import tpu_sc as plsc

EPS = 1e-12
CHUNK = 32
NBW = 3
NBP = 2


def _emb_rmsnorm_sc(ids, pids, word_table, pos_table, norm_weight):
    NW_, n_ch, _ = ids.shape
    D = word_table.shape[1]
    info = plsc.get_sparse_core_info()
    NC, NS, L = info.num_cores, info.num_subcores, info.num_lanes
    NW = NC * NS
    assert NW_ == NW
    N = NW * n_ch * CHUNK
    per_w = n_ch * CHUNK
    nvec = D // L

    mesh = plsc.VectorSubcoreMesh(core_axis_name="c", subcore_axis_name="s")

    @functools.partial(
        pl.kernel,
        mesh=mesh,
        out_type=jax.ShapeDtypeStruct((N, D), jnp.float32),
        compiler_params=pltpu.CompilerParams(needs_layout_passes=False),
        scratch_types=[
            pltpu.VMEM((n_ch, CHUNK), jnp.int32),
            pltpu.VMEM((n_ch, CHUNK), jnp.int32),
            pltpu.VMEM((NBW, CHUNK, D), jnp.float32),
            pltpu.VMEM((NBP, CHUNK, D), jnp.float32),
            pltpu.VMEM((D,), jnp.float32),
            pltpu.SemaphoreType.DMA((NBW,)),
            pltpu.SemaphoreType.DMA((NBP,)),
            pltpu.SemaphoreType.DMA((NBW,)),
        ],
    )
    def emb_kernel(ids_hbm, pid_hbm, wt_hbm, pt_hbm, nw_hbm, out_hbm,
                   widx, pidxv, wbuf, pbuf, nwv, semw, semp, semo):
        wid = lax.axis_index("s") * NC + lax.axis_index("c")
        base = wid * per_w

        pltpu.sync_copy(nw_hbm, nwv)
        pltpu.sync_copy(ids_hbm.at[wid], widx)
        pltpu.sync_copy(pid_hbm.at[wid], pidxv)

        def w_desc(c):
            b = lax.rem(c, NBW)
            return pltpu.make_async_copy(wt_hbm.at[widx.at[c]], wbuf.at[b],
                                         semw.at[b])

        def p_desc(c):
            b = lax.rem(c, NBP)
            return pltpu.make_async_copy(pt_hbm.at[pidxv.at[c]], pbuf.at[b],
                                         semp.at[b])

        def out_desc(c):
            b = lax.rem(c, NBW)
            return pltpu.make_async_copy(
                wbuf.at[b], out_hbm.at[pl.ds(base + c * CHUNK, CHUNK)],
                semo.at[b])

        def gather(c):
            w_desc(c).start()
            p_desc(c).start()

        def compute(c):
            b = lax.rem(c, NBW)
            bp = lax.rem(c, NBP)

            @plsc.parallel_loop(0, CHUNK, unroll=2)
            def body(t):
                accs = [jnp.zeros((L,), jnp.float32) for _ in range(4)]
                for j in range(nvec):
                    sl = pl.ds(j * L, L)
                    v = wbuf[b, t, sl] + pbuf[bp, t, sl]
                    wbuf[b, t, sl] = v * nwv[sl]
                    accs[j & 3] = accs[j & 3] + v * v
                total = jnp.sum((accs[0] + accs[1]) + (accs[2] + accs[3]))
                dv = jnp.broadcast_to(total * (1.0 / D) + EPS, (L,))
                bits = plsc.bitcast(dv, jnp.int32)
                magic = jnp.full((L,), 0x5F3759DF, dtype=jnp.int32)
                one = jnp.full((L,), 1, dtype=jnp.int32)
                y = plsc.bitcast(magic - lax.shift_right_logical(bits, one),
                                 jnp.float32)
                for _ in range(2):
                    y = y * (1.5 - 0.5 * dv * y * y)
                for j in range(nvec):
                    sl = pl.ds(j * L, L)
                    wbuf[b, t, sl] = wbuf[b, t, sl] * y

        gather(jnp.int32(0))

        def body(c, carry):
            @pl.when(c + 1 < n_ch)
            def _():
                @pl.when(c >= 2)
                def _():
                    out_desc(c - 2).wait()
                gather(c + 1)

            w_desc(c).wait()
            p_desc(c).wait()
            compute(c)
            out_desc(c).start()
            return carry

        lax.fori_loop(0, n_ch, body, 0)
        for c in range(max(n_ch - NBW, 0), n_ch):
            out_desc(jnp.int32(c)).wait()

    return emb_kernel(ids, pids, word_table, pos_table, norm_weight)


def kernel(input_ids, position_idcs, word_table, pos_table, norm_weight):
    B, S = input_ids.shape
    D = word_table.shape[1]
    N = B * S
    NW = 32
    per_w = N // NW
    n_ch = per_w // CHUNK
    ids = input_ids.reshape(NW, n_ch, CHUNK).astype(jnp.int32)
    pids = position_idcs.reshape(NW, n_ch, CHUNK).astype(jnp.int32)
    out = _emb_rmsnorm_sc(ids, pids, word_table.astype(jnp.float32),
                          pos_table.astype(jnp.float32),
                          norm_weight.astype(jnp.float32))
    return out.reshape(B, S, D)


# speedup vs baseline: 1.0660x; 1.0660x over previous
import functools

import jax
import jax.numpy as jnp
from jax import lax
from jax.experimental import pallas as pl
from jax.experimental.pallas import tpu as pltpu
from jax.experimental.pallas import tpu_sc as plsc

EPS = 1e-12
CHUNK = 32
NBUF = 2


def _emb_rmsnorm_sc(ids, pids, word_table, pos_table, norm_weight):
    NW_, n_ch, _ = ids.shape
    D = word_table.shape[1]
    info = plsc.get_sparse_core_info()
    NC, NS, L = info.num_cores, info.num_subcores, info.num_lanes
    NW = NC * NS
    assert NW_ == NW
    N = NW * n_ch * CHUNK
    per_w = n_ch * CHUNK
    nvec = D // L

    mesh = plsc.VectorSubcoreMesh(core_axis_name="c", subcore_axis_name="s")

    @functools.partial(
        pl.kernel,
        mesh=mesh,
        out_type=jax.ShapeDtypeStruct((N, D), jnp.float32),
        compiler_params=pltpu.CompilerParams(needs_layout_passes=False),
        scratch_types=[
            pltpu.VMEM((n_ch, CHUNK), jnp.int32),
            pltpu.VMEM((n_ch, CHUNK), jnp.int32),
            pltpu.VMEM((NBUF, CHUNK, D), jnp.float32),
            pltpu.VMEM((NBUF, CHUNK, D), jnp.float32),
            pltpu.VMEM((D,), jnp.float32),
            pltpu.SemaphoreType.DMA((NBUF,)),
            pltpu.SemaphoreType.DMA((NBUF,)),
            pltpu.SemaphoreType.DMA((NBUF,)),
        ],
    )
    def emb_kernel(ids_hbm, pid_hbm, wt_hbm, pt_hbm, nw_hbm, out_hbm,
                   widx, pidxv, wbuf, pbuf, nwv, semw, semp, semo):
        wid = lax.axis_index("s") * NC + lax.axis_index("c")
        base = wid * per_w

        pltpu.sync_copy(nw_hbm, nwv)
        pltpu.sync_copy(ids_hbm.at[wid], widx)
        pltpu.sync_copy(pid_hbm.at[wid], pidxv)

        def w_desc(c):
            b = lax.rem(c, NBUF)
            return pltpu.make_async_copy(wt_hbm.at[widx.at[c]], wbuf.at[b],
                                         semw.at[b])

        def p_desc(c):
            b = lax.rem(c, NBUF)
            return pltpu.make_async_copy(pt_hbm.at[pidxv.at[c]], pbuf.at[b],
                                         semp.at[b])

        def out_desc(c):
            b = lax.rem(c, NBUF)
            return pltpu.make_async_copy(
                wbuf.at[b], out_hbm.at[pl.ds(base + c * CHUNK, CHUNK)],
                semo.at[b])

        def gather(c):
            w_desc(c).start()
            p_desc(c).start()

        def compute(c):
            b = lax.rem(c, NBUF)

            @plsc.parallel_loop(0, CHUNK, unroll=2)
            def body(t):
                accs = [jnp.zeros((L,), jnp.float32) for _ in range(4)]
                for j in range(nvec):
                    sl = pl.ds(j * L, L)
                    v = wbuf[b, t, sl] + pbuf[b, t, sl]
                    wbuf[b, t, sl] = v * nwv[sl]
                    accs[j & 3] = accs[j & 3] + v * v
                total = jnp.sum((accs[0] + accs[1]) + (accs[2] + accs[3]))
                dv = jnp.broadcast_to(total * (1.0 / D) + EPS, (L,))
                bits = plsc.bitcast(dv, jnp.int32)
                magic = jnp.full((L,), 0x5F3759DF, dtype=jnp.int32)
                one = jnp.full((L,), 1, dtype=jnp.int32)
                y = plsc.bitcast(magic - lax.shift_right_logical(bits, one),
                                 jnp.float32)
                for _ in range(2):
                    y = y * (1.5 - 0.5 * dv * y * y)
                for j in range(nvec):
                    sl = pl.ds(j * L, L)
                    wbuf[b, t, sl] = wbuf[b, t, sl] * y

        gather(jnp.int32(0))

        def body(c, carry):
            @pl.when(c + 1 < n_ch)
            def _():
                @pl.when(c >= 1)
                def _():
                    out_desc(c - 1).wait()
                gather(c + 1)

            w_desc(c).wait()
            p_desc(c).wait()
            compute(c)
            out_desc(c).start()
            return carry

        lax.fori_loop(0, n_ch, body, 0)
        for c in range(max(n_ch - NBUF, 0), n_ch):
            out_desc(jnp.int32(c)).wait()

    return emb_kernel(ids, pids, word_table, pos_table, norm_weight)


def kernel(input_ids, position_idcs, word_table, pos_table, norm_weight):
    B, S = input_ids.shape
    D = word_table.shape[1]
    N = B * S
    NW = 32
    per_w = N // NW
    n_ch = per_w // CHUNK
    ids = input_ids.reshape(NW, n_ch, CHUNK).astype(jnp.int32)
    pids = position_idcs.reshape(NW, n_ch, CHUNK).astype(jnp.int32)
    out = _emb_rmsnorm_sc(ids, pids, word_table.astype(jnp.float32),
                          pos_table.astype(jnp.float32),
                          norm_weight.astype(jnp.float32))
    return out.reshape(B, S, D)


# speedup vs baseline: 1.1749x; 1.1021x over previous
import functools

import jax
import jax.numpy as jnp
from jax import lax
from jax.experimental import pallas as pl
from jax.experimental.pallas import tpu as pltpu
from jax.experimental.pallas import tpu_sc as plsc

EPS = 1e-12
CHUNK = 16
NBUF = 4


def _emb_rmsnorm_sc(ids, pids, word_table, pos_table, norm_weight):
    NW_, n_ch, _ = ids.shape
    D = word_table.shape[1]
    info = plsc.get_sparse_core_info()
    NC, NS, L = info.num_cores, info.num_subcores, info.num_lanes
    NW = NC * NS
    assert NW_ == NW
    N = NW * n_ch * CHUNK
    per_w = n_ch * CHUNK
    nvec = D // L

    mesh = plsc.VectorSubcoreMesh(core_axis_name="c", subcore_axis_name="s")

    @functools.partial(
        pl.kernel,
        mesh=mesh,
        out_type=jax.ShapeDtypeStruct((N, D), jnp.float32),
        compiler_params=pltpu.CompilerParams(needs_layout_passes=False),
        scratch_types=[
            pltpu.VMEM((n_ch, CHUNK), jnp.int32),
            pltpu.VMEM((n_ch, CHUNK), jnp.int32),
            pltpu.VMEM((NBUF, CHUNK, D), jnp.float32),
            pltpu.VMEM((NBUF, CHUNK, D), jnp.float32),
            pltpu.VMEM((D,), jnp.float32),
            pltpu.SemaphoreType.DMA((NBUF,)),
            pltpu.SemaphoreType.DMA((NBUF,)),
            pltpu.SemaphoreType.DMA((NBUF,)),
        ],
    )
    def emb_kernel(ids_hbm, pid_hbm, wt_hbm, pt_hbm, nw_hbm, out_hbm,
                   widx, pidxv, wbuf, pbuf, nwv, semw, semp, semo):
        wid = lax.axis_index("s") * NC + lax.axis_index("c")
        base = wid * per_w

        pltpu.sync_copy(nw_hbm, nwv)
        pltpu.sync_copy(ids_hbm.at[wid], widx)
        pltpu.sync_copy(pid_hbm.at[wid], pidxv)

        def w_desc(c):
            b = lax.rem(c, NBUF)
            return pltpu.make_async_copy(wt_hbm.at[widx.at[c]], wbuf.at[b],
                                         semw.at[b])

        def p_desc(c):
            b = lax.rem(c, NBUF)
            return pltpu.make_async_copy(pt_hbm.at[pidxv.at[c]], pbuf.at[b],
                                         semp.at[b])

        def out_desc(c):
            b = lax.rem(c, NBUF)
            return pltpu.make_async_copy(
                wbuf.at[b], out_hbm.at[pl.ds(base + c * CHUNK, CHUNK)],
                semo.at[b])

        def gather(c):
            w_desc(c).start()
            p_desc(c).start()

        def compute(c):
            b = lax.rem(c, NBUF)

            @plsc.parallel_loop(0, CHUNK, unroll=2)
            def body(t):
                accs = [jnp.zeros((L,), jnp.float32) for _ in range(4)]
                for j in range(nvec):
                    sl = pl.ds(j * L, L)
                    v = wbuf[b, t, sl] + pbuf[b, t, sl]
                    wbuf[b, t, sl] = v * nwv[sl]
                    accs[j & 3] = accs[j & 3] + v * v
                total = jnp.sum((accs[0] + accs[1]) + (accs[2] + accs[3]))
                dv = jnp.broadcast_to(total * (1.0 / D) + EPS, (L,))
                bits = plsc.bitcast(dv, jnp.int32)
                magic = jnp.full((L,), 0x5F3759DF, dtype=jnp.int32)
                one = jnp.full((L,), 1, dtype=jnp.int32)
                y = plsc.bitcast(magic - lax.shift_right_logical(bits, one),
                                 jnp.float32)
                for _ in range(2):
                    y = y * (1.5 - 0.5 * dv * y * y)
                for j in range(nvec):
                    sl = pl.ds(j * L, L)
                    wbuf[b, t, sl] = wbuf[b, t, sl] * y

        gather(jnp.int32(0))

        def body(c, carry):
            @pl.when(c + 1 < n_ch)
            def _():
                @pl.when(c >= NBUF - 1)
                def _():
                    out_desc(c + 1 - NBUF).wait()
                gather(c + 1)

            w_desc(c).wait()
            p_desc(c).wait()
            compute(c)
            out_desc(c).start()
            return carry

        lax.fori_loop(0, n_ch, body, 0)
        for c in range(max(n_ch - NBUF, 0), n_ch):
            out_desc(jnp.int32(c)).wait()

    return emb_kernel(ids, pids, word_table, pos_table, norm_weight)


def kernel(input_ids, position_idcs, word_table, pos_table, norm_weight):
    B, S = input_ids.shape
    D = word_table.shape[1]
    N = B * S
    NW = 32
    per_w = N // NW
    n_ch = per_w // CHUNK
    ids = input_ids.reshape(NW, n_ch, CHUNK).astype(jnp.int32)
    pids = position_idcs.reshape(NW, n_ch, CHUNK).astype(jnp.int32)
    out = _emb_rmsnorm_sc(ids, pids, word_table.astype(jnp.float32),
                          pos_table.astype(jnp.float32),
                          norm_weight.astype(jnp.float32))
    return out.reshape(B, S, D)


# speedup vs baseline: 1.1828x; 1.0068x over previous
import functools

import jax
import jax.numpy as jnp
from jax import lax
from jax.experimental import pallas as pl
from jax.experimental.pallas import tpu as pltpu
from jax.experimental.pallas import tpu_sc as plsc

EPS = 1e-12
CHUNK = 8
NBUF = 8


def _emb_rmsnorm_sc(ids, pids, word_table, pos_table, norm_weight):
    NW_, n_ch, _ = ids.shape
    D = word_table.shape[1]
    info = plsc.get_sparse_core_info()
    NC, NS, L = info.num_cores, info.num_subcores, info.num_lanes
    NW = NC * NS
    assert NW_ == NW
    N = NW * n_ch * CHUNK
    per_w = n_ch * CHUNK
    nvec = D // L

    mesh = plsc.VectorSubcoreMesh(core_axis_name="c", subcore_axis_name="s")

    @functools.partial(
        pl.kernel,
        mesh=mesh,
        out_type=jax.ShapeDtypeStruct((N, D), jnp.float32),
        compiler_params=pltpu.CompilerParams(needs_layout_passes=False),
        scratch_types=[
            pltpu.VMEM((n_ch, CHUNK), jnp.int32),
            pltpu.VMEM((n_ch, CHUNK), jnp.int32),
            pltpu.VMEM((NBUF, CHUNK, D), jnp.float32),
            pltpu.VMEM((NBUF, CHUNK, D), jnp.float32),
            pltpu.VMEM((D,), jnp.float32),
            pltpu.SemaphoreType.DMA((NBUF,)),
            pltpu.SemaphoreType.DMA((NBUF,)),
            pltpu.SemaphoreType.DMA((NBUF,)),
        ],
    )
    def emb_kernel(ids_hbm, pid_hbm, wt_hbm, pt_hbm, nw_hbm, out_hbm,
                   widx, pidxv, wbuf, pbuf, nwv, semw, semp, semo):
        wid = lax.axis_index("s") * NC + lax.axis_index("c")
        base = wid * per_w

        pltpu.sync_copy(nw_hbm, nwv)
        pltpu.sync_copy(ids_hbm.at[wid], widx)
        pltpu.sync_copy(pid_hbm.at[wid], pidxv)

        def w_desc(c):
            b = lax.rem(c, NBUF)
            return pltpu.make_async_copy(wt_hbm.at[widx.at[c]], wbuf.at[b],
                                         semw.at[b])

        def p_desc(c):
            b = lax.rem(c, NBUF)
            return pltpu.make_async_copy(pt_hbm.at[pidxv.at[c]], pbuf.at[b],
                                         semp.at[b])

        def out_desc(c):
            b = lax.rem(c, NBUF)
            return pltpu.make_async_copy(
                wbuf.at[b], out_hbm.at[pl.ds(base + c * CHUNK, CHUNK)],
                semo.at[b])

        def gather(c):
            w_desc(c).start()
            p_desc(c).start()

        def compute(c):
            b = lax.rem(c, NBUF)

            @plsc.parallel_loop(0, CHUNK, unroll=2)
            def body(t):
                accs = [jnp.zeros((L,), jnp.float32) for _ in range(4)]
                for j in range(nvec):
                    sl = pl.ds(j * L, L)
                    v = wbuf[b, t, sl] + pbuf[b, t, sl]
                    wbuf[b, t, sl] = v * nwv[sl]
                    accs[j & 3] = accs[j & 3] + v * v
                total = jnp.sum((accs[0] + accs[1]) + (accs[2] + accs[3]))
                dv = jnp.broadcast_to(total * (1.0 / D) + EPS, (L,))
                bits = plsc.bitcast(dv, jnp.int32)
                magic = jnp.full((L,), 0x5F3759DF, dtype=jnp.int32)
                one = jnp.full((L,), 1, dtype=jnp.int32)
                y = plsc.bitcast(magic - lax.shift_right_logical(bits, one),
                                 jnp.float32)
                for _ in range(2):
                    y = y * (1.5 - 0.5 * dv * y * y)
                for j in range(nvec):
                    sl = pl.ds(j * L, L)
                    wbuf[b, t, sl] = wbuf[b, t, sl] * y

        gather(jnp.int32(0))

        def body(c, carry):
            @pl.when(c + 1 < n_ch)
            def _():
                @pl.when(c >= NBUF - 1)
                def _():
                    out_desc(c + 1 - NBUF).wait()
                gather(c + 1)

            w_desc(c).wait()
            p_desc(c).wait()
            compute(c)
            out_desc(c).start()
            return carry

        lax.fori_loop(0, n_ch, body, 0)
        for c in range(max(n_ch - NBUF, 0), n_ch):
            out_desc(jnp.int32(c)).wait()

    return emb_kernel(ids, pids, word_table, pos_table, norm_weight)


def kernel(input_ids, position_idcs, word_table, pos_table, norm_weight):
    B, S = input_ids.shape
    D = word_table.shape[1]
    N = B * S
    NW = 32
    per_w = N // NW
    n_ch = per_w // CHUNK
    ids = input_ids.reshape(NW, n_ch, CHUNK).astype(jnp.int32)
    pids = position_idcs.reshape(NW, n_ch, CHUNK).astype(jnp.int32)
    out = _emb_rmsnorm_sc(ids, pids, word_table.astype(jnp.float32),
                          pos_table.astype(jnp.float32),
                          norm_weight.astype(jnp.float32))
    return out.reshape(B, S, D)
